# Initial kernel scaffold; baseline (speedup 1.0000x reference)
import jax
import jax.numpy as jnp
from jax.experimental import pallas as pl


def kernel(action, feature, hidden, cell, ctx, emb, W_ih, W_hh, b_ih, b_hh, W_out, b_out):
    raise NotImplementedError("write your pallas kernel here")


# speedup vs baseline: 1.0976x; 1.0976x over previous
import jax
import jax.numpy as jnp
from jax.experimental import pallas as pl
from jax.experimental.pallas import tpu as pltpu

B = 8
E = 256
H = 2048
FEAT = 2048
V_IN = 16
V_OUT = 16

TILE = 256


def _lstm_step_kernel(action_ref, feature_ref, hidden_ref, cell_ref, emb_ref,
                      wih_ref, whh_ref, bih_ref, bhh_ref, wout_ref, bout_ref,
                      h1_ref, c1_ref, logit_ref):
    i = pl.program_id(0)

    a = action_ref[...]
    onehot = (a == jax.lax.broadcasted_iota(jnp.int32, (B, V_IN), 1))
    x_emb = jnp.dot(onehot.astype(jnp.float32), emb_ref[...],
                    preferred_element_type=jnp.float32)
    x = jnp.concatenate([x_emb, feature_ref[...]], axis=1)
    h0 = hidden_ref[...]

    wih = wih_ref[...].reshape(4 * TILE, E + FEAT)
    whh = whh_ref[...].reshape(4 * TILE, H)

    pre = (jnp.dot(x, wih.T, preferred_element_type=jnp.float32) +
           jnp.dot(h0, whh.T, preferred_element_type=jnp.float32))

    def gate(k):
        return (pre[:, k * TILE:(k + 1) * TILE] +
                bih_ref[k, 0, :] + bhh_ref[k, 0, :])

    ig = jax.nn.sigmoid(gate(0))
    fg = jax.nn.sigmoid(gate(1))
    gg = jnp.tanh(gate(2))
    og = jax.nn.sigmoid(gate(3))

    c0 = cell_ref[...]
    c1 = fg * c0 + ig * gg
    h1 = og * jnp.tanh(c1)
    c1_ref[...] = c1
    h1_ref[...] = h1

    part = jnp.dot(h1, wout_ref[...].T, preferred_element_type=jnp.float32)

    @pl.when(i == 0)
    def _():
        logit_ref[...] = part + bout_ref[...]

    @pl.when(i > 0)
    def _():
        logit_ref[...] += part


def kernel(action, feature, hidden, cell, ctx, emb, W_ih, W_hh, b_ih, b_hh,
           W_out, b_out):
    del ctx
    n_tiles = H // TILE

    h0 = hidden.reshape(B, H)
    c0 = cell.reshape(B, H)
    wih4 = W_ih.reshape(4, H, E + FEAT)
    whh4 = W_hh.reshape(4, H, H)
    bih4 = b_ih.reshape(4, 1, H)
    bhh4 = b_hh.reshape(4, 1, H)
    bout2 = b_out.reshape(1, V_OUT)
    action32 = action.astype(jnp.int32)

    grid_spec = pltpu.PrefetchScalarGridSpec(
        num_scalar_prefetch=0,
        grid=(n_tiles,),
        in_specs=[
            pl.BlockSpec((B, 1), lambda i: (0, 0)),
            pl.BlockSpec((B, FEAT), lambda i: (0, 0)),
            pl.BlockSpec((B, H), lambda i: (0, 0)),
            pl.BlockSpec((B, TILE), lambda i: (0, i)),
            pl.BlockSpec((V_IN, E), lambda i: (0, 0)),
            pl.BlockSpec((4, TILE, E + FEAT), lambda i: (0, i, 0)),
            pl.BlockSpec((4, TILE, H), lambda i: (0, i, 0)),
            pl.BlockSpec((4, 1, TILE), lambda i: (0, 0, i)),
            pl.BlockSpec((4, 1, TILE), lambda i: (0, 0, i)),
            pl.BlockSpec((V_OUT, TILE), lambda i: (0, i)),
            pl.BlockSpec((1, V_OUT), lambda i: (0, 0)),
        ],
        out_specs=[
            pl.BlockSpec((B, TILE), lambda i: (0, i)),
            pl.BlockSpec((B, TILE), lambda i: (0, i)),
            pl.BlockSpec((B, V_OUT), lambda i: (0, 0)),
        ],
    )

    h1, c1, logit = pl.pallas_call(
        _lstm_step_kernel,
        grid_spec=grid_spec,
        out_shape=[
            jax.ShapeDtypeStruct((B, H), jnp.float32),
            jax.ShapeDtypeStruct((B, H), jnp.float32),
            jax.ShapeDtypeStruct((B, V_OUT), jnp.float32),
        ],
        compiler_params=pltpu.CompilerParams(
            dimension_semantics=("arbitrary",),
        ),
    )(action32, feature, h0, c0, emb, wih4, whh4, bih4, bhh4, W_out, bout2)

    return (h1[None, :, :], c1[None, :, :], logit)


# speedup vs baseline: 1.1623x; 1.0589x over previous
import jax
import jax.numpy as jnp
from jax.experimental import pallas as pl
from jax.experimental.pallas import tpu as pltpu

B = 8
E = 256
H = 2048
FEAT = 2048
V_IN = 16
V_OUT = 16

TILE = 128


def _lstm_step_kernel(action_ref, feature_ref, hidden_ref, cell_ref, emb_ref,
                      wih_ref, whh_ref, bih_ref, bhh_ref, wout_ref, bout_ref,
                      h1_ref, c1_ref, logit_ref):
    i = pl.program_id(0)

    a = action_ref[...]
    onehot = (a == jax.lax.broadcasted_iota(jnp.int32, (B, V_IN), 1))
    x_emb = jnp.dot(onehot.astype(jnp.float32), emb_ref[...],
                    preferred_element_type=jnp.float32)
    x = jnp.concatenate([x_emb, feature_ref[...]], axis=1)
    h0 = hidden_ref[...]

    wih = wih_ref[...].reshape(4 * TILE, E + FEAT)
    whh = whh_ref[...].reshape(4 * TILE, H)

    pre = (jnp.dot(x, wih.T, preferred_element_type=jnp.float32) +
           jnp.dot(h0, whh.T, preferred_element_type=jnp.float32))

    def gate(k):
        return (pre[:, k * TILE:(k + 1) * TILE] +
                bih_ref[k, 0, :] + bhh_ref[k, 0, :])

    ig = jax.nn.sigmoid(gate(0))
    fg = jax.nn.sigmoid(gate(1))
    gg = jnp.tanh(gate(2))
    og = jax.nn.sigmoid(gate(3))

    c0 = cell_ref[...]
    c1 = fg * c0 + ig * gg
    h1 = og * jnp.tanh(c1)
    c1_ref[...] = c1
    h1_ref[...] = h1

    part = jnp.dot(h1, wout_ref[...].T, preferred_element_type=jnp.float32)

    @pl.when(i == 0)
    def _():
        logit_ref[...] = part + bout_ref[...]

    @pl.when(i > 0)
    def _():
        logit_ref[...] += part


def kernel(action, feature, hidden, cell, ctx, emb, W_ih, W_hh, b_ih, b_hh,
           W_out, b_out):
    del ctx
    n_tiles = H // TILE

    h0 = hidden.reshape(B, H)
    c0 = cell.reshape(B, H)
    wih4 = W_ih.reshape(4, H, E + FEAT)
    whh4 = W_hh.reshape(4, H, H)
    bih4 = b_ih.reshape(4, 1, H)
    bhh4 = b_hh.reshape(4, 1, H)
    bout2 = b_out.reshape(1, V_OUT)
    action32 = action.astype(jnp.int32)

    grid_spec = pltpu.PrefetchScalarGridSpec(
        num_scalar_prefetch=0,
        grid=(n_tiles,),
        in_specs=[
            pl.BlockSpec((B, 1), lambda i: (0, 0)),
            pl.BlockSpec((B, FEAT), lambda i: (0, 0)),
            pl.BlockSpec((B, H), lambda i: (0, 0)),
            pl.BlockSpec((B, TILE), lambda i: (0, i)),
            pl.BlockSpec((V_IN, E), lambda i: (0, 0)),
            pl.BlockSpec((4, TILE, E + FEAT), lambda i: (0, i, 0)),
            pl.BlockSpec((4, TILE, H), lambda i: (0, i, 0)),
            pl.BlockSpec((4, 1, TILE), lambda i: (0, 0, i)),
            pl.BlockSpec((4, 1, TILE), lambda i: (0, 0, i)),
            pl.BlockSpec((V_OUT, TILE), lambda i: (0, i)),
            pl.BlockSpec((1, V_OUT), lambda i: (0, 0)),
        ],
        out_specs=[
            pl.BlockSpec((B, TILE), lambda i: (0, i)),
            pl.BlockSpec((B, TILE), lambda i: (0, i)),
            pl.BlockSpec((B, V_OUT), lambda i: (0, 0)),
        ],
    )

    h1, c1, logit = pl.pallas_call(
        _lstm_step_kernel,
        grid_spec=grid_spec,
        out_shape=[
            jax.ShapeDtypeStruct((B, H), jnp.float32),
            jax.ShapeDtypeStruct((B, H), jnp.float32),
            jax.ShapeDtypeStruct((B, V_OUT), jnp.float32),
        ],
        compiler_params=pltpu.CompilerParams(
            dimension_semantics=("arbitrary",),
        ),
    )(action32, feature, h0, c0, emb, wih4, whh4, bih4, bhh4, W_out, bout2)

    return (h1[None, :, :], c1[None, :, :], logit)
